# Initial kernel scaffold; baseline (speedup 1.0000x reference)
import functools

import jax
import jax.numpy as jnp
from jax import lax
from jax.experimental import pallas as pl
from jax.experimental.pallas import tpu as pltpu
from jax.experimental.pallas import tpu_sc as plsc

N = 10000
E = 320000
D = 128

NC = 2
NS = 16
NW = NC * NS
EPW = E // NW
CH = 80
NCH = EPW // CH
RPS = N // NS

_mesh = plsc.VectorSubcoreMesh(core_axis_name="c", subcore_axis_name="s",
                               num_cores=NC)


def _deg_body(dst_hbm, ones_hbm, z_hbm, out_hbm, dst_v, ones_v, deg_sh):
    c = lax.axis_index("c")
    s = lax.axis_index("s")
    wid = s * NC + c
    pltpu.sync_copy(z_hbm.at[pl.ds(s * RPS, RPS)],
                    deg_sh.at[pl.ds(s * RPS, RPS)])
    pltpu.sync_copy(ones_hbm, ones_v)
    pltpu.sync_copy(dst_hbm.at[wid], dst_v)
    plsc.subcore_barrier()

    def chunk(j, carry):
        pltpu.sync_copy(ones_v, deg_sh.at[dst_v.at[j]], add=True)
        return carry

    lax.fori_loop(0, NCH, chunk, 0)
    plsc.subcore_barrier()
    pltpu.sync_copy(deg_sh.at[pl.ds(s * RPS, RPS)],
                    out_hbm.at[c, pl.ds(s * RPS, RPS)])


@jax.jit
def _deg_pass(dst3, ones8, zeros8):
    return pl.kernel(
        _deg_body,
        out_type=jax.ShapeDtypeStruct((NC, N, 8), jnp.float32),
        mesh=_mesh,
        scratch_types=[
            pltpu.VMEM((NCH, CH), jnp.int32),
            pltpu.VMEM((CH, 8), jnp.float32),
            pltpu.VMEM_SHARED((N, 8), jnp.float32),
        ],
    )(dst3, ones8, zeros8)


def _msg_body(src_hbm, dst_hbm, u_hbm, z_hbm, out_hbm,
              src_v, dst_v, rows_v, acc_sh, sem):
    c = lax.axis_index("c")
    s = lax.axis_index("s")
    wid = s * NC + c
    pltpu.sync_copy(z_hbm.at[pl.ds(s * RPS, RPS)],
                    acc_sh.at[pl.ds(s * RPS, RPS)])
    pltpu.sync_copy(src_hbm.at[wid], src_v)
    pltpu.sync_copy(dst_hbm.at[wid], dst_v)
    plsc.subcore_barrier()

    def chunk(j, carry):
        pltpu.async_copy(u_hbm.at[src_v.at[j]], rows_v, sem).wait()
        pltpu.sync_copy(rows_v, acc_sh.at[dst_v.at[j]], add=True)
        return carry

    lax.fori_loop(0, NCH, chunk, 0)
    plsc.subcore_barrier()
    pltpu.sync_copy(acc_sh.at[pl.ds(s * RPS, RPS)],
                    out_hbm.at[c, pl.ds(s * RPS, RPS)])


@jax.jit
def _msg_pass(src3, dst3, u, zeros128):
    return pl.kernel(
        _msg_body,
        out_type=jax.ShapeDtypeStruct((NC, N, D), jnp.float32),
        mesh=_mesh,
        scratch_types=[
            pltpu.VMEM((NCH, CH), jnp.int32),
            pltpu.VMEM((NCH, CH), jnp.int32),
            pltpu.VMEM((CH, D), jnp.float32),
            pltpu.VMEM_SHARED((N, D), jnp.float32),
            pltpu.SemaphoreType.DMA,
        ],
    )(src3, dst3, u, zeros128)


R = 1000


def _leaky(v):
    return jnp.where(v > 0, v, 0.1 * v)


def _enc_body(degp_ref, x_ref, We_ref, be_ref, Wg1_ref, dinv_ref, u1_ref):
    deg = degp_ref[0, :, 0:1] + degp_ref[1, :, 0:1] + 1.0
    dinv = lax.rsqrt(deg)
    h0 = _leaky(jnp.dot(x_ref[...], We_ref[...],
                        preferred_element_type=jnp.float32) + be_ref[...])
    u1 = dinv * jnp.dot(h0, Wg1_ref[...], preferred_element_type=jnp.float32)
    dinv_ref[...] = jnp.broadcast_to(dinv, (R, D))
    u1_ref[...] = u1


def _mid_body(acc_ref, u_ref, dinv_ref, b_ref, W_ref, unext_ref):
    tot = acc_ref[0] + acc_ref[1] + u_ref[...]
    h = _leaky(dinv_ref[...] * tot + b_ref[...])
    unext_ref[...] = dinv_ref[...] * jnp.dot(
        h, W_ref[...], preferred_element_type=jnp.float32)


def _dec_body(acc_ref, u_ref, dinv_ref, b_ref, Wd_ref, bd_ref, out_ref):
    tot = acc_ref[0] + acc_ref[1] + u_ref[...]
    h = _leaky(dinv_ref[...] * tot + b_ref[...])
    out_ref[...] = jnp.dot(h, Wd_ref[...],
                           preferred_element_type=jnp.float32) + bd_ref[...]


_row_spec = pl.BlockSpec((R, D), lambda i: (i, 0))
_acc_spec = pl.BlockSpec((NC, R, D), lambda i: (0, i, 0))
_w_spec = pl.BlockSpec((D, D), lambda i: (0, 0))
_b_spec = pl.BlockSpec((1, D), lambda i: (0, 0))


@jax.jit
def _enc_pass(degp, x, W_enc, b_enc, W_g1):
    return pl.pallas_call(
        _enc_body,
        grid=(N // R,),
        in_specs=[
            pl.BlockSpec((NC, R, 8), lambda i: (0, i, 0)),
            _row_spec, _w_spec, _b_spec, _w_spec,
        ],
        out_specs=[_row_spec, _row_spec],
        out_shape=[jax.ShapeDtypeStruct((N, D), jnp.float32),
                   jax.ShapeDtypeStruct((N, D), jnp.float32)],
    )(degp, x, W_enc, b_enc, W_g1)


@jax.jit
def _mid_pass(acc, u, dinv, b, W):
    return pl.pallas_call(
        _mid_body,
        grid=(N // R,),
        in_specs=[_acc_spec, _row_spec, _row_spec, _b_spec, _w_spec],
        out_specs=_row_spec,
        out_shape=jax.ShapeDtypeStruct((N, D), jnp.float32),
    )(acc, u, dinv, b, W)


@jax.jit
def _dec_pass(acc, u, dinv, b, W_dec, b_dec):
    return pl.pallas_call(
        _dec_body,
        grid=(N // R,),
        in_specs=[_acc_spec, _row_spec, _row_spec, _b_spec, _w_spec, _b_spec],
        out_specs=_row_spec,
        out_shape=jax.ShapeDtypeStruct((N, D), jnp.float32),
    )(acc, u, dinv, b, W_dec, b_dec)


def kernel(x, g, W_enc, b_enc, W_g1, b_g1, W_g2, b_g2, W_dec, b_dec):
    src3 = g[0].reshape(NW, NCH, CH)
    dst3 = g[1].reshape(NW, NCH, CH)
    ones8 = jnp.ones((CH, 8), jnp.float32)
    zeros8 = jnp.zeros((N, 8), jnp.float32)
    zeros128 = jnp.zeros((N, D), jnp.float32)
    b_enc2 = b_enc.reshape(1, D)
    b_g12 = b_g1.reshape(1, D)
    b_g22 = b_g2.reshape(1, D)
    b_dec2 = b_dec.reshape(1, D)

    degp = _deg_pass(dst3, ones8, zeros8)
    dinv, u1 = _enc_pass(degp, x, W_enc, b_enc2, W_g1)
    acc1 = _msg_pass(src3, dst3, u1, zeros128)
    u2 = _mid_pass(acc1, u1, dinv, b_g12, W_g2)
    acc2 = _msg_pass(src3, dst3, u2, zeros128)
    out = _dec_pass(acc2, u2, dinv, b_g22, W_dec, b_dec2)
    return out


# speedup vs baseline: 19.3347x; 19.3347x over previous
import functools

import jax
import jax.numpy as jnp
from jax import lax
from jax.experimental import pallas as pl
from jax.experimental.pallas import tpu as pltpu
from jax.experimental.pallas import tpu_sc as plsc

N = 10000
E = 320000
D = 128

NC = 2
NS = 16
NW = NC * NS
EPW = E // NW
CH = 80
NCH = EPW // CH
NP = 10240
RPS = NP // NS

_mesh = plsc.VectorSubcoreMesh(core_axis_name="c", subcore_axis_name="s",
                               num_cores=NC)


def _deg_body(dst_hbm, ones_hbm, z_hbm, out_hbm, dst_v, ones_v, deg_sh):
    c = lax.axis_index("c")
    s = lax.axis_index("s")
    wid = s * NC + c
    pltpu.sync_copy(z_hbm.at[pl.ds(s * RPS, RPS)],
                    deg_sh.at[pl.ds(s * RPS, RPS)])
    pltpu.sync_copy(ones_hbm, ones_v)
    pltpu.sync_copy(dst_hbm.at[wid], dst_v)
    plsc.subcore_barrier()

    def chunk(j, carry):
        pltpu.sync_copy(ones_v, deg_sh.at[dst_v.at[j]], add=True)
        return carry

    lax.fori_loop(0, NCH, chunk, 0)
    plsc.subcore_barrier()
    pltpu.sync_copy(deg_sh.at[pl.ds(s * RPS, RPS)],
                    out_hbm.at[c, pl.ds(s * RPS, RPS)])


@jax.jit
def _deg_pass(dst3, ones8, zeros8):
    return pl.kernel(
        _deg_body,
        out_type=jax.ShapeDtypeStruct((NC, NP, 8), jnp.float32),
        mesh=_mesh,
        scratch_types=[
            pltpu.VMEM((NCH, CH), jnp.int32),
            pltpu.VMEM((CH, 8), jnp.float32),
            pltpu.VMEM_SHARED((NP, 8), jnp.float32),
        ],
    )(dst3, ones8, zeros8)


def _msg_body(src_hbm, dst_hbm, u_hbm, z_hbm, out_hbm,
              src_v, dst_v, rows_v, acc_sh, sem):
    c = lax.axis_index("c")
    s = lax.axis_index("s")
    wid = s * NC + c
    pltpu.sync_copy(z_hbm.at[pl.ds(s * RPS, RPS)],
                    acc_sh.at[pl.ds(s * RPS, RPS)])
    pltpu.sync_copy(src_hbm.at[wid], src_v)
    pltpu.sync_copy(dst_hbm.at[wid], dst_v)
    plsc.subcore_barrier()

    def chunk(j, carry):
        pltpu.async_copy(u_hbm.at[src_v.at[j]], rows_v, sem).wait()
        pltpu.sync_copy(rows_v, acc_sh.at[dst_v.at[j]], add=True)
        return carry

    lax.fori_loop(0, NCH, chunk, 0)
    plsc.subcore_barrier()
    pltpu.sync_copy(acc_sh.at[pl.ds(s * RPS, RPS)],
                    out_hbm.at[c, pl.ds(s * RPS, RPS)])


@jax.jit
def _msg_pass(src3, dst3, u, zeros128):
    return pl.kernel(
        _msg_body,
        out_type=jax.ShapeDtypeStruct((NC, NP, D), jnp.float32),
        mesh=_mesh,
        scratch_types=[
            pltpu.VMEM((NCH, CH), jnp.int32),
            pltpu.VMEM((NCH, CH), jnp.int32),
            pltpu.VMEM((CH, D), jnp.float32),
            pltpu.VMEM_SHARED((NP, D), jnp.float32),
            pltpu.SemaphoreType.DMA,
        ],
    )(src3, dst3, u, zeros128)


R = 1000


def _leaky(v):
    return jnp.where(v > 0, v, 0.1 * v)


def _enc_body(degp_ref, x_ref, We_ref, be_ref, Wg1_ref, dinv_ref, u1_ref):
    deg = degp_ref[0, :, 0:1] + degp_ref[1, :, 0:1] + 1.0
    dinv = lax.rsqrt(deg)
    h0 = _leaky(jnp.dot(x_ref[...], We_ref[...],
                        preferred_element_type=jnp.float32) + be_ref[...])
    u1 = dinv * jnp.dot(h0, Wg1_ref[...], preferred_element_type=jnp.float32)
    dinv_ref[...] = jnp.broadcast_to(dinv, (R, D))
    u1_ref[...] = u1


def _mid_body(acc_ref, u_ref, dinv_ref, b_ref, W_ref, unext_ref):
    tot = acc_ref[0] + acc_ref[1] + u_ref[...]
    h = _leaky(dinv_ref[...] * tot + b_ref[...])
    unext_ref[...] = dinv_ref[...] * jnp.dot(
        h, W_ref[...], preferred_element_type=jnp.float32)


def _dec_body(acc_ref, u_ref, dinv_ref, b_ref, Wd_ref, bd_ref, out_ref):
    tot = acc_ref[0] + acc_ref[1] + u_ref[...]
    h = _leaky(dinv_ref[...] * tot + b_ref[...])
    out_ref[...] = jnp.dot(h, Wd_ref[...],
                           preferred_element_type=jnp.float32) + bd_ref[...]


_row_spec = pl.BlockSpec((R, D), lambda i: (i, 0))
_acc_spec = pl.BlockSpec((NC, R, D), lambda i: (0, i, 0))
_w_spec = pl.BlockSpec((D, D), lambda i: (0, 0))
_b_spec = pl.BlockSpec((1, D), lambda i: (0, 0))


@jax.jit
def _enc_pass(degp, x, W_enc, b_enc, W_g1):
    return pl.pallas_call(
        _enc_body,
        grid=(N // R,),
        in_specs=[
            pl.BlockSpec((NC, R, 8), lambda i: (0, i, 0)),
            _row_spec, _w_spec, _b_spec, _w_spec,
        ],
        out_specs=[_row_spec, _row_spec],
        out_shape=[jax.ShapeDtypeStruct((N, D), jnp.float32),
                   jax.ShapeDtypeStruct((N, D), jnp.float32)],
    )(degp, x, W_enc, b_enc, W_g1)


@jax.jit
def _mid_pass(acc, u, dinv, b, W):
    return pl.pallas_call(
        _mid_body,
        grid=(N // R,),
        in_specs=[_acc_spec, _row_spec, _row_spec, _b_spec, _w_spec],
        out_specs=_row_spec,
        out_shape=jax.ShapeDtypeStruct((N, D), jnp.float32),
    )(acc, u, dinv, b, W)


@jax.jit
def _dec_pass(acc, u, dinv, b, W_dec, b_dec):
    return pl.pallas_call(
        _dec_body,
        grid=(N // R,),
        in_specs=[_acc_spec, _row_spec, _row_spec, _b_spec, _w_spec, _b_spec],
        out_specs=_row_spec,
        out_shape=jax.ShapeDtypeStruct((N, D), jnp.float32),
    )(acc, u, dinv, b, W_dec, b_dec)


def kernel(x, g, W_enc, b_enc, W_g1, b_g1, W_g2, b_g2, W_dec, b_dec):
    src3 = g[0].reshape(NW, NCH, CH)
    dst3 = g[1].reshape(NW, NCH, CH)
    ones8 = jnp.ones((CH, 8), jnp.float32)
    zeros8 = jnp.zeros((NP, 8), jnp.float32)
    zeros128 = jnp.zeros((NP, D), jnp.float32)
    b_enc2 = b_enc.reshape(1, D)
    b_g12 = b_g1.reshape(1, D)
    b_g22 = b_g2.reshape(1, D)
    b_dec2 = b_dec.reshape(1, D)

    degp = _deg_pass(dst3, ones8, zeros8)
    dinv, u1 = _enc_pass(degp, x, W_enc, b_enc2, W_g1)
    acc1 = _msg_pass(src3, dst3, u1, zeros128)
    u2 = _mid_pass(acc1, u1, dinv, b_g12, W_g2)
    acc2 = _msg_pass(src3, dst3, u2, zeros128)
    out = _dec_pass(acc2, u2, dinv, b_g22, W_dec, b_dec2)
    return out
